# baseline (speedup 1.0000x reference)
import functools

import jax
import jax.numpy as jnp
from jax import lax
from jax.experimental import pallas as pl
from jax.experimental.pallas import tpu as pltpu
from jax.experimental.pallas import tpu_sc as plsc

_N_EDGES = 160000
_D = 256
_N_NODES = 10000
_HALF = 128
_LANES = 16
_IDXROWS = _N_EDGES // 128
_N_TILES = 16
_ROWS_PER = _IDXROWS // _N_TILES
_ROWS_REM = _IDXROWS - _ROWS_PER * _N_TILES
_NBLK = _N_NODES // 8
_NBLK_PER = _NBLK // _N_TILES
_NBLK_REM = _NBLK - _NBLK_PER * _N_TILES
_MAIN_ROWS = _NBLK_PER * 8
_CHUNK = 104
_NCHUNK = _MAIN_ROWS // _CHUNK
_SCALE = 0.25


def _sc_body(
    x_hbm, idx_hbm, out_hbm, acc,
    idx_a, idx_b, data_a, data_b, sem_a, sem_b, ssem_a, ssem_b,
):
    c = lax.axis_index("c")
    s = lax.axis_index("s")
    col0 = c * _HALF

    node0 = (s * _NBLK_PER + jnp.minimum(s, _NBLK_REM)) * 8
    has_extra_nodes = s < _NBLK_REM
    extra_node0 = node0 + _MAIN_ROWS

    def _x_slice(row):
        return x_hbm.at[pl.ds(row * 128, 128), pl.ds(col0, _HALF)]

    def _idx_slice(row):
        return idx_hbm.at[pl.ds(row * 128, 128)]

    def _issue(row, data, idxb, sem):
        pltpu.async_copy(_x_slice(row), data, sem)
        pltpu.async_copy(_idx_slice(row), idxb, sem)

    def _wait(row, data, idxb, sem):
        pltpu.make_async_copy(_x_slice(row), data, sem).wait()
        pltpu.make_async_copy(_idx_slice(row), idxb, sem).wait()

    def _zero_row(i, carry):
        for q in range(_HALF // _LANES):
            data_a[i, pl.ds(q * _LANES, _LANES)] = jnp.zeros(
                (_LANES,), jnp.float32
            )
        return carry

    lax.fori_loop(0, _CHUNK, _zero_row, 0)
    for k in range(_NCHUNK):
        pltpu.sync_copy(
            data_a.at[pl.ds(0, _CHUNK)],
            acc.at[pl.ds(node0 + k * _CHUNK, _CHUNK)],
        )

    @pl.when(has_extra_nodes)
    def _():
        pltpu.sync_copy(
            data_a.at[pl.ds(0, 8)], acc.at[pl.ds(extra_node0, 8)]
        )

    base = s * _ROWS_PER + jnp.minimum(s, _ROWS_REM)
    has_extra_rows = s < _ROWS_REM
    plsc.subcore_barrier()

    cnt = _ROWS_PER + (s < _ROWS_REM).astype(jnp.int32)

    def _scat_start(data, idxb, ssem):
        return pltpu.async_copy(data, acc.at[idxb], ssem, add=True)

    def _scat_wait(data, idxb, ssem):
        pltpu.make_async_copy(data, acc.at[idxb], ssem).wait()

    _issue(base, data_a, idx_a, sem_a)

    def _pair(j, carry):
        b = base + 2 * j
        _wait(b, data_a, idx_a, sem_a)
        _scat_start(data_a, idx_a, ssem_a)

        @pl.when(j > 0)
        def _():
            _scat_wait(data_b, idx_b, ssem_b)

        _issue(b + 1, data_b, idx_b, sem_b)
        _wait(b + 1, data_b, idx_b, sem_b)
        _scat_start(data_b, idx_b, ssem_b)
        _scat_wait(data_a, idx_a, ssem_a)

        @pl.when(b + 2 < cnt + base)
        def _():
            _issue(b + 2, data_a, idx_a, sem_a)

        return carry

    lax.fori_loop(0, _ROWS_PER // 2, _pair, 0)
    _scat_wait(data_b, idx_b, ssem_b)

    @pl.when(has_extra_rows)
    def _():
        b = base + _ROWS_PER
        _wait(b, data_a, idx_a, sem_a)
        pltpu.sync_copy(data_a, acc.at[idx_a], add=True)

    plsc.subcore_barrier()

    bufs = [(data_a, sem_a), (data_b, sem_b)]

    def _acc_slice(k):
        return acc.at[pl.ds(node0 + k * _CHUNK, _CHUNK)]

    def _scale_rows(buf, n):
        def _scale_row(i, carry):
            for q in range(_HALF // _LANES):
                sl = pl.ds(q * _LANES, _LANES)
                buf[i, sl] = buf[i, sl] * _SCALE
            return carry

        lax.fori_loop(0, n, _scale_row, 0)

    pltpu.async_copy(_acc_slice(0), data_a.at[pl.ds(0, _CHUNK)], sem_a)
    for k in range(_NCHUNK):
        buf, sem = bufs[k % 2]
        nbuf, nsem = bufs[(k + 1) % 2]
        pltpu.make_async_copy(_acc_slice(k), buf.at[pl.ds(0, _CHUNK)], sem).wait()
        if k + 1 < _NCHUNK:
            pltpu.async_copy(
                _acc_slice(k + 1), nbuf.at[pl.ds(0, _CHUNK)], nsem
            )
        elif True:
            @pl.when(has_extra_nodes)
            def _():
                pltpu.async_copy(
                    acc.at[pl.ds(extra_node0, 8)], nbuf.at[pl.ds(0, 8)], nsem
                )
        _scale_rows(buf, _CHUNK)
        pltpu.sync_copy(
            buf.at[pl.ds(0, _CHUNK)],
            out_hbm.at[pl.ds(node0 + k * _CHUNK, _CHUNK), pl.ds(col0, _HALF)],
        )

    @pl.when(has_extra_nodes)
    def _():
        tbuf, tsem = bufs[_NCHUNK % 2]
        pltpu.make_async_copy(
            acc.at[pl.ds(extra_node0, 8)], tbuf.at[pl.ds(0, 8)], tsem
        ).wait()
        _scale_rows(tbuf, 8)
        pltpu.sync_copy(
            tbuf.at[pl.ds(0, 8)],
            out_hbm.at[pl.ds(extra_node0, 8), pl.ds(col0, _HALF)],
        )


@jax.jit
def _scatter_sc(x, idx3d):
    mesh = plsc.VectorSubcoreMesh(core_axis_name="c", subcore_axis_name="s")
    f = functools.partial(
        pl.kernel,
        out_type=jax.ShapeDtypeStruct((_N_NODES, _D), jnp.float32),
        mesh=mesh,
        scratch_types=[
            pltpu.VMEM_SHARED((_N_NODES, _HALF), jnp.float32),
            pltpu.VMEM((128,), jnp.int32),
            pltpu.VMEM((128,), jnp.int32),
            pltpu.VMEM((128, _HALF), jnp.float32),
            pltpu.VMEM((128, _HALF), jnp.float32),
            pltpu.SemaphoreType.DMA,
            pltpu.SemaphoreType.DMA,
            pltpu.SemaphoreType.DMA,
            pltpu.SemaphoreType.DMA,
        ],
    )(_sc_body)
    return f(x, idx3d)


def kernel(x, index, dim, dim_size):
    idx = jnp.clip(
        index.astype(jnp.int32) + jnp.asarray(dim, jnp.int32),
        0,
        jnp.asarray(dim_size, jnp.int32) - 1,
    )
    return _scatter_sc(x, idx)


# speedup vs baseline: 1.1442x; 1.1442x over previous
import functools

import jax
import jax.numpy as jnp
from jax import lax
from jax.experimental import pallas as pl
from jax.experimental.pallas import tpu as pltpu
from jax.experimental.pallas import tpu_sc as plsc

_N_EDGES = 160000
_D = 256
_N_NODES = 10000
_HALF = 128
_LANES = 16
_IDXROWS = _N_EDGES // 128
_N_TILES = 16
_ROWS_PER = _IDXROWS // _N_TILES
_ROWS_REM = _IDXROWS - _ROWS_PER * _N_TILES
_NBLK = _N_NODES // 8
_NBLK_PER = _NBLK // _N_TILES
_NBLK_REM = _NBLK - _NBLK_PER * _N_TILES
_MAIN_ROWS = _NBLK_PER * 8
_CHUNK = 104
_NCHUNK = _MAIN_ROWS // _CHUNK
_SCALE = 0.25


def _sc_body(
    x_hbm, idx_hbm, out_hbm, acc,
    idx_a, idx_b, data_a, data_b, sem_a, sem_b, ssem_a, ssem_b,
):
    c = lax.axis_index("c")
    s = lax.axis_index("s")
    col0 = c * _HALF

    node0 = (s * _NBLK_PER + jnp.minimum(s, _NBLK_REM)) * 8
    has_extra_nodes = s < _NBLK_REM
    extra_node0 = node0 + _MAIN_ROWS

    def _x_slice(row):
        return x_hbm.at[pl.ds(row * 128, 128), pl.ds(col0, _HALF)]

    def _idx_slice(row):
        return idx_hbm.at[pl.ds(row * 128, 128)]

    def _issue(row, data, idxb, sem):
        pltpu.async_copy(_x_slice(row), data, sem)
        pltpu.async_copy(_idx_slice(row), idxb, sem)

    def _wait(row, data, idxb, sem):
        pltpu.make_async_copy(_x_slice(row), data, sem).wait()
        pltpu.make_async_copy(_idx_slice(row), idxb, sem).wait()

    base = s * _ROWS_PER + jnp.minimum(s, _ROWS_REM)
    has_extra_rows = s < _ROWS_REM
    _issue(base, data_a, idx_a, sem_a)

    def _zero_row(i, carry):
        for q in range(_HALF // _LANES):
            data_b[i, pl.ds(q * _LANES, _LANES)] = jnp.zeros(
                (_LANES,), jnp.float32
            )
        return carry

    lax.fori_loop(0, _CHUNK, _zero_row, 0)
    for k in range(_NCHUNK):
        pltpu.sync_copy(
            data_b.at[pl.ds(0, _CHUNK)],
            acc.at[pl.ds(node0 + k * _CHUNK, _CHUNK)],
        )

    @pl.when(has_extra_nodes)
    def _():
        pltpu.sync_copy(
            data_b.at[pl.ds(0, 8)], acc.at[pl.ds(extra_node0, 8)]
        )

    plsc.subcore_barrier()

    cnt = _ROWS_PER + (s < _ROWS_REM).astype(jnp.int32)

    def _pair(j, carry):
        b = base + 2 * j
        _issue(b + 1, data_b, idx_b, sem_b)
        _wait(b, data_a, idx_a, sem_a)
        pltpu.sync_copy(data_a, acc.at[idx_a], add=True)

        @pl.when(2 * j + 2 < cnt)
        def _():
            _issue(b + 2, data_a, idx_a, sem_a)

        _wait(b + 1, data_b, idx_b, sem_b)
        pltpu.sync_copy(data_b, acc.at[idx_b], add=True)
        return carry

    lax.fori_loop(0, _ROWS_PER // 2, _pair, 0)

    @pl.when(has_extra_rows)
    def _():
        b = base + _ROWS_PER
        _wait(b, data_a, idx_a, sem_a)
        pltpu.sync_copy(data_a, acc.at[idx_a], add=True)

    plsc.subcore_barrier()

    bufs = [(data_a, sem_a), (data_b, sem_b)]

    def _acc_slice(k):
        return acc.at[pl.ds(node0 + k * _CHUNK, _CHUNK)]

    def _scale_rows(buf, n):
        def _scale_row(i, carry):
            for q in range(_HALF // _LANES):
                sl = pl.ds(q * _LANES, _LANES)
                buf[i, sl] = buf[i, sl] * _SCALE
            return carry

        lax.fori_loop(0, n, _scale_row, 0)

    pltpu.async_copy(_acc_slice(0), data_a.at[pl.ds(0, _CHUNK)], sem_a)
    for k in range(_NCHUNK):
        buf, sem = bufs[k % 2]
        nbuf, nsem = bufs[(k + 1) % 2]
        pltpu.make_async_copy(_acc_slice(k), buf.at[pl.ds(0, _CHUNK)], sem).wait()
        if k + 1 < _NCHUNK:
            pltpu.async_copy(
                _acc_slice(k + 1), nbuf.at[pl.ds(0, _CHUNK)], nsem
            )
        elif True:
            @pl.when(has_extra_nodes)
            def _():
                pltpu.async_copy(
                    acc.at[pl.ds(extra_node0, 8)], nbuf.at[pl.ds(0, 8)], nsem
                )
        _scale_rows(buf, _CHUNK)
        pltpu.sync_copy(
            buf.at[pl.ds(0, _CHUNK)],
            out_hbm.at[pl.ds(node0 + k * _CHUNK, _CHUNK), pl.ds(col0, _HALF)],
        )

    @pl.when(has_extra_nodes)
    def _():
        tbuf, tsem = bufs[_NCHUNK % 2]
        pltpu.make_async_copy(
            acc.at[pl.ds(extra_node0, 8)], tbuf.at[pl.ds(0, 8)], tsem
        ).wait()
        _scale_rows(tbuf, 8)
        pltpu.sync_copy(
            tbuf.at[pl.ds(0, 8)],
            out_hbm.at[pl.ds(extra_node0, 8), pl.ds(col0, _HALF)],
        )


@jax.jit
def _scatter_sc(x, idx3d):
    mesh = plsc.VectorSubcoreMesh(core_axis_name="c", subcore_axis_name="s")
    f = functools.partial(
        pl.kernel,
        out_type=jax.ShapeDtypeStruct((_N_NODES, _D), jnp.float32),
        mesh=mesh,
        scratch_types=[
            pltpu.VMEM_SHARED((_N_NODES, _HALF), jnp.float32),
            pltpu.VMEM((128,), jnp.int32),
            pltpu.VMEM((128,), jnp.int32),
            pltpu.VMEM((128, _HALF), jnp.float32),
            pltpu.VMEM((128, _HALF), jnp.float32),
            pltpu.SemaphoreType.DMA,
            pltpu.SemaphoreType.DMA,
            pltpu.SemaphoreType.DMA,
            pltpu.SemaphoreType.DMA,
        ],
    )(_sc_body)
    return f(x, idx3d)


def kernel(x, index, dim, dim_size):
    idx = jnp.clip(
        index.astype(jnp.int32) + jnp.asarray(dim, jnp.int32),
        0,
        jnp.asarray(dim_size, jnp.int32) - 1,
    )
    return _scatter_sc(x, idx)


# speedup vs baseline: 1.2123x; 1.0595x over previous
import functools

import jax
import jax.numpy as jnp
from jax import lax
from jax.experimental import pallas as pl
from jax.experimental.pallas import tpu as pltpu
from jax.experimental.pallas import tpu_sc as plsc

_N_EDGES = 160000
_D = 256
_N_NODES = 10000
_HALF = 128
_LANES = 16
_IDXROWS = _N_EDGES // 128
_N_TILES = 16
_ROWS_PER = _IDXROWS // _N_TILES
_ROWS_REM = _IDXROWS - _ROWS_PER * _N_TILES
_NSLOTS = 3
_NGROUPS = _ROWS_PER // _NSLOTS
_NBLK = _N_NODES // 8
_NBLK_PER = _NBLK // _N_TILES
_NBLK_REM = _NBLK - _NBLK_PER * _N_TILES
_MAIN_ROWS = _NBLK_PER * 8
_CHUNK = 104
_NCHUNK = _MAIN_ROWS // _CHUNK
_SCALE = 0.25


def _sc_body(
    x_hbm, idx_hbm, out_hbm, acc,
    idx_a, idx_b, idx_c, data_a, data_b, data_c, sem_a, sem_b, sem_c,
):
    c = lax.axis_index("c")
    s = lax.axis_index("s")
    col0 = c * _HALF

    node0 = (s * _NBLK_PER + jnp.minimum(s, _NBLK_REM)) * 8
    has_extra_nodes = s < _NBLK_REM
    extra_node0 = node0 + _MAIN_ROWS

    slots = [
        (data_a, idx_a, sem_a),
        (data_b, idx_b, sem_b),
        (data_c, idx_c, sem_c),
    ]

    def _x_slice(row):
        return x_hbm.at[pl.ds(row * 128, 128), pl.ds(col0, _HALF)]

    def _idx_slice(row):
        return idx_hbm.at[pl.ds(row * 128, 128)]

    def _issue(row, data, idxb, sem):
        pltpu.async_copy(_x_slice(row), data, sem)
        pltpu.async_copy(_idx_slice(row), idxb, sem)

    def _wait(row, data, idxb, sem):
        pltpu.make_async_copy(_x_slice(row), data, sem).wait()
        pltpu.make_async_copy(_idx_slice(row), idxb, sem).wait()

    base = s * _ROWS_PER + jnp.minimum(s, _ROWS_REM)
    has_extra_rows = s < _ROWS_REM
    cnt = _ROWS_PER + (s < _ROWS_REM).astype(jnp.int32)

    _issue(base, data_a, idx_a, sem_a)
    _issue(base + 1, data_b, idx_b, sem_b)

    def _zero_row(i, carry):
        for q in range(_HALF // _LANES):
            data_c[i, pl.ds(q * _LANES, _LANES)] = jnp.zeros(
                (_LANES,), jnp.float32
            )
        return carry

    lax.fori_loop(0, _CHUNK, _zero_row, 0)
    for k in range(_NCHUNK):
        pltpu.sync_copy(
            data_c.at[pl.ds(0, _CHUNK)],
            acc.at[pl.ds(node0 + k * _CHUNK, _CHUNK)],
        )

    @pl.when(has_extra_nodes)
    def _():
        pltpu.sync_copy(
            data_c.at[pl.ds(0, 8)], acc.at[pl.ds(extra_node0, 8)]
        )

    _issue(base + 2, data_c, idx_c, sem_c)
    plsc.subcore_barrier()

    def _group(j, carry):
        t0 = base + _NSLOTS * j
        for r, (data, idxb, sem) in enumerate(slots):
            t = t0 + r
            _wait(t, data, idxb, sem)
            pltpu.sync_copy(data, acc.at[idxb], add=True)

            @pl.when(t + _NSLOTS < base + cnt)
            def _():
                _issue(t + _NSLOTS, data, idxb, sem)

        return carry

    lax.fori_loop(0, _NGROUPS, _group, 0)

    @pl.when(has_extra_rows)
    def _():
        b = base + _ROWS_PER
        _wait(b, data_a, idx_a, sem_a)
        pltpu.sync_copy(data_a, acc.at[idx_a], add=True)

    plsc.subcore_barrier()

    bufs = [(data_a, sem_a), (data_b, sem_b)]

    def _acc_slice(k):
        return acc.at[pl.ds(node0 + k * _CHUNK, _CHUNK)]

    def _scale_rows(buf, n):
        def _scale_row(i, carry):
            for q in range(_HALF // _LANES):
                sl = pl.ds(q * _LANES, _LANES)
                buf[i, sl] = buf[i, sl] * _SCALE
            return carry

        lax.fori_loop(0, n, _scale_row, 0)

    pltpu.async_copy(_acc_slice(0), data_a.at[pl.ds(0, _CHUNK)], sem_a)
    for k in range(_NCHUNK):
        buf, sem = bufs[k % 2]
        nbuf, nsem = bufs[(k + 1) % 2]
        pltpu.make_async_copy(_acc_slice(k), buf.at[pl.ds(0, _CHUNK)], sem).wait()
        if k + 1 < _NCHUNK:
            pltpu.async_copy(
                _acc_slice(k + 1), nbuf.at[pl.ds(0, _CHUNK)], nsem
            )
        else:
            @pl.when(has_extra_nodes)
            def _():
                pltpu.async_copy(
                    acc.at[pl.ds(extra_node0, 8)], nbuf.at[pl.ds(0, 8)], nsem
                )
        _scale_rows(buf, _CHUNK)
        pltpu.sync_copy(
            buf.at[pl.ds(0, _CHUNK)],
            out_hbm.at[pl.ds(node0 + k * _CHUNK, _CHUNK), pl.ds(col0, _HALF)],
        )

    @pl.when(has_extra_nodes)
    def _():
        tbuf, tsem = bufs[_NCHUNK % 2]
        pltpu.make_async_copy(
            acc.at[pl.ds(extra_node0, 8)], tbuf.at[pl.ds(0, 8)], tsem
        ).wait()
        _scale_rows(tbuf, 8)
        pltpu.sync_copy(
            tbuf.at[pl.ds(0, 8)],
            out_hbm.at[pl.ds(extra_node0, 8), pl.ds(col0, _HALF)],
        )


@jax.jit
def _scatter_sc(x, idx1d):
    mesh = plsc.VectorSubcoreMesh(core_axis_name="c", subcore_axis_name="s")
    f = functools.partial(
        pl.kernel,
        out_type=jax.ShapeDtypeStruct((_N_NODES, _D), jnp.float32),
        mesh=mesh,
        scratch_types=[
            pltpu.VMEM_SHARED((_N_NODES, _HALF), jnp.float32),
            pltpu.VMEM((128,), jnp.int32),
            pltpu.VMEM((128,), jnp.int32),
            pltpu.VMEM((128,), jnp.int32),
            pltpu.VMEM((128, _HALF), jnp.float32),
            pltpu.VMEM((128, _HALF), jnp.float32),
            pltpu.VMEM((128, _HALF), jnp.float32),
            pltpu.SemaphoreType.DMA,
            pltpu.SemaphoreType.DMA,
            pltpu.SemaphoreType.DMA,
        ],
    )(_sc_body)
    return f(x, idx1d)


def kernel(x, index, dim, dim_size):
    idx = jnp.clip(
        index.astype(jnp.int32) + jnp.asarray(dim, jnp.int32),
        0,
        jnp.asarray(dim_size, jnp.int32) - 1,
    )
    return _scatter_sc(x, idx)


# speedup vs baseline: 1.2128x; 1.0004x over previous
import functools

import jax
import jax.numpy as jnp
from jax import lax
from jax.experimental import pallas as pl
from jax.experimental.pallas import tpu as pltpu
from jax.experimental.pallas import tpu_sc as plsc

_N_EDGES = 160000
_D = 256
_N_NODES = 10000
_HALF = 128
_LANES = 16
_IDXROWS = _N_EDGES // 128
_N_TILES = 16
_ROWS_PER = _IDXROWS // _N_TILES
_ROWS_REM = _IDXROWS - _ROWS_PER * _N_TILES
_NSLOTS = 3
_NGROUPS = _ROWS_PER // _NSLOTS
_NBLK = _N_NODES // 8
_NBLK_PER = _NBLK // _N_TILES
_NBLK_REM = _NBLK - _NBLK_PER * _N_TILES
_MAIN_ROWS = _NBLK_PER * 8
_CHUNK = 104
_NCHUNK = _MAIN_ROWS // _CHUNK
_SCALE = 0.25


def _sc_body(
    x_hbm, idx_hbm, out_hbm, acc,
    idx_a, idx_b, idx_c, data_a, data_b, data_c,
    sem_a, sem_b, sem_c, sem_z, sem_wa, sem_wb,
):
    c = lax.axis_index("c")
    s = lax.axis_index("s")
    col0 = c * _HALF

    node0 = (s * _NBLK_PER + jnp.minimum(s, _NBLK_REM)) * 8
    has_extra_nodes = s < _NBLK_REM
    extra_node0 = node0 + _MAIN_ROWS

    slots = [
        (data_a, idx_a, sem_a),
        (data_b, idx_b, sem_b),
        (data_c, idx_c, sem_c),
    ]

    def _x_slice(row):
        return x_hbm.at[pl.ds(row * 128, 128), pl.ds(col0, _HALF)]

    def _idx_slice(row):
        return idx_hbm.at[pl.ds(row * 128, 128)]

    def _issue(row, data, idxb, sem):
        pltpu.async_copy(_x_slice(row), data, sem)
        pltpu.async_copy(_idx_slice(row), idxb, sem)

    def _wait(row, data, idxb, sem):
        pltpu.make_async_copy(_x_slice(row), data, sem).wait()
        pltpu.make_async_copy(_idx_slice(row), idxb, sem).wait()

    base = s * _ROWS_PER + jnp.minimum(s, _ROWS_REM)
    has_extra_rows = s < _ROWS_REM
    cnt = _ROWS_PER + (s < _ROWS_REM).astype(jnp.int32)

    _issue(base, data_a, idx_a, sem_a)
    _issue(base + 1, data_b, idx_b, sem_b)

    def _zero_row(i, carry):
        for q in range(_HALF // _LANES):
            data_c[i, pl.ds(q * _LANES, _LANES)] = jnp.zeros(
                (_LANES,), jnp.float32
            )
        return carry

    lax.fori_loop(0, _CHUNK, _zero_row, 0)
    for k in range(_NCHUNK):
        pltpu.async_copy(
            data_c.at[pl.ds(0, _CHUNK)],
            acc.at[pl.ds(node0 + k * _CHUNK, _CHUNK)],
            sem_z,
        )

    @pl.when(has_extra_nodes)
    def _():
        pltpu.async_copy(
            data_c.at[pl.ds(0, 8)], acc.at[pl.ds(extra_node0, 8)], sem_z
        )

    for k in range(_NCHUNK):
        pltpu.make_async_copy(
            data_c.at[pl.ds(0, _CHUNK)],
            acc.at[pl.ds(node0 + k * _CHUNK, _CHUNK)],
            sem_z,
        ).wait()

    @pl.when(has_extra_nodes)
    def _():
        pltpu.make_async_copy(
            data_c.at[pl.ds(0, 8)], acc.at[pl.ds(extra_node0, 8)], sem_z
        ).wait()

    _issue(base + 2, data_c, idx_c, sem_c)
    plsc.subcore_barrier()

    def _group(j, carry):
        t0 = base + _NSLOTS * j
        for r, (data, idxb, sem) in enumerate(slots):
            t = t0 + r
            _wait(t, data, idxb, sem)
            pltpu.sync_copy(data, acc.at[idxb], add=True)

            @pl.when(t + _NSLOTS < base + cnt)
            def _():
                _issue(t + _NSLOTS, data, idxb, sem)

        return carry

    lax.fori_loop(0, _NGROUPS, _group, 0)

    @pl.when(has_extra_rows)
    def _():
        b = base + _ROWS_PER
        _wait(b, data_a, idx_a, sem_a)
        pltpu.sync_copy(data_a, acc.at[idx_a], add=True)

    plsc.subcore_barrier()

    bufs = [(data_a, sem_a), (data_b, sem_b)]

    def _acc_slice(k):
        return acc.at[pl.ds(node0 + k * _CHUNK, _CHUNK)]

    def _scale_rows(buf, n):
        def _scale_row(i, carry):
            for q in range(_HALF // _LANES):
                sl = pl.ds(q * _LANES, _LANES)
                buf[i, sl] = buf[i, sl] * _SCALE
            return carry

        lax.fori_loop(0, n, _scale_row, 0)

    wsems = [sem_wa, sem_wb]

    def _out_slice(k):
        return out_hbm.at[
            pl.ds(node0 + k * _CHUNK, _CHUNK), pl.ds(col0, _HALF)
        ]

    pltpu.async_copy(_acc_slice(0), data_a.at[pl.ds(0, _CHUNK)], sem_a)
    for k in range(_NCHUNK):
        buf, sem = bufs[k % 2]
        nbuf, nsem = bufs[(k + 1) % 2]
        wsem, nwsem = wsems[k % 2], wsems[(k + 1) % 2]
        pltpu.make_async_copy(_acc_slice(k), buf.at[pl.ds(0, _CHUNK)], sem).wait()
        if k + 1 < _NCHUNK:
            if k >= 1:
                pltpu.make_async_copy(
                    nbuf.at[pl.ds(0, _CHUNK)], _out_slice(k - 1), nwsem
                ).wait()
            pltpu.async_copy(
                _acc_slice(k + 1), nbuf.at[pl.ds(0, _CHUNK)], nsem
            )
        else:
            pltpu.make_async_copy(
                nbuf.at[pl.ds(0, _CHUNK)], _out_slice(k - 1), nwsem
            ).wait()

            @pl.when(has_extra_nodes)
            def _():
                pltpu.async_copy(
                    acc.at[pl.ds(extra_node0, 8)], nbuf.at[pl.ds(0, 8)], nsem
                )
        _scale_rows(buf, _CHUNK)
        pltpu.async_copy(buf.at[pl.ds(0, _CHUNK)], _out_slice(k), wsem)

    pltpu.make_async_copy(
        bufs[(_NCHUNK - 1) % 2][0].at[pl.ds(0, _CHUNK)],
        _out_slice(_NCHUNK - 1),
        wsems[(_NCHUNK - 1) % 2],
    ).wait()

    @pl.when(has_extra_nodes)
    def _():
        tbuf, tsem = bufs[_NCHUNK % 2]
        pltpu.make_async_copy(
            acc.at[pl.ds(extra_node0, 8)], tbuf.at[pl.ds(0, 8)], tsem
        ).wait()
        _scale_rows(tbuf, 8)
        pltpu.sync_copy(
            tbuf.at[pl.ds(0, 8)],
            out_hbm.at[pl.ds(extra_node0, 8), pl.ds(col0, _HALF)],
        )


@jax.jit
def _scatter_sc(x, idx1d):
    mesh = plsc.VectorSubcoreMesh(core_axis_name="c", subcore_axis_name="s")
    f = functools.partial(
        pl.kernel,
        out_type=jax.ShapeDtypeStruct((_N_NODES, _D), jnp.float32),
        mesh=mesh,
        scratch_types=[
            pltpu.VMEM_SHARED((_N_NODES, _HALF), jnp.float32),
            pltpu.VMEM((128,), jnp.int32),
            pltpu.VMEM((128,), jnp.int32),
            pltpu.VMEM((128,), jnp.int32),
            pltpu.VMEM((128, _HALF), jnp.float32),
            pltpu.VMEM((128, _HALF), jnp.float32),
            pltpu.VMEM((128, _HALF), jnp.float32),
            pltpu.SemaphoreType.DMA,
            pltpu.SemaphoreType.DMA,
            pltpu.SemaphoreType.DMA,
            pltpu.SemaphoreType.DMA,
            pltpu.SemaphoreType.DMA,
            pltpu.SemaphoreType.DMA,
        ],
    )(_sc_body)
    return f(x, idx1d)


def kernel(x, index, dim, dim_size):
    idx = jnp.clip(
        index.astype(jnp.int32) + jnp.asarray(dim, jnp.int32),
        0,
        jnp.asarray(dim_size, jnp.int32) - 1,
    )
    return _scatter_sc(x, idx)
